# baseline (speedup 1.0000x reference)
import functools

import jax
import jax.numpy as jnp
from jax import lax
from jax.experimental import pallas as pl
from jax.experimental.pallas import tpu as pltpu
from jax.experimental.pallas import tpu_sc as plsc

_NC = 2
_NS = 16
_NW = _NC * _NS
_CHUNK = 32
_NCHUNK = 4


def _sc_copy(x_hbm, o_hbm, buf, in_sems, out_sems):
    wid = lax.axis_index("s") * _NC + lax.axis_index("c")
    base = wid * (_CHUNK * _NCHUNK)

    def in_copy(j, b):
        return pltpu.make_async_copy(
            x_hbm.at[pl.ds(base + j * _CHUNK, _CHUNK)], buf.at[b], in_sems.at[b]
        )

    def out_copy(j, b):
        return pltpu.make_async_copy(
            buf.at[b], o_hbm.at[pl.ds(base + j * _CHUNK, _CHUNK)], out_sems.at[b]
        )

    ins = [in_copy(j, j % 2) for j in range(_NCHUNK)]
    outs = [out_copy(j, j % 2) for j in range(_NCHUNK)]
    ins[0].start()
    for j in range(_NCHUNK):
        if j + 1 < _NCHUNK:
            if j - 1 >= 0:
                outs[j - 1].wait()
            ins[j + 1].start()
        ins[j].wait()
        outs[j].start()
    outs[_NCHUNK - 2].wait()
    outs[_NCHUNK - 1].wait()


def kernel(batch):
    B, F = batch.shape
    mesh = plsc.VectorSubcoreMesh(core_axis_name="c", subcore_axis_name="s")
    k = functools.partial(
        pl.kernel,
        mesh=mesh,
        out_type=jax.ShapeDtypeStruct((B, F), batch.dtype),
        scratch_types=[
            pltpu.VMEM((2, _CHUNK, F), batch.dtype),
            pltpu.SemaphoreType.DMA((2,)),
            pltpu.SemaphoreType.DMA((2,)),
        ],
    )(_sc_copy)
    return k(batch)


# speedup vs baseline: 1.0347x; 1.0347x over previous
import functools

import jax
import jax.numpy as jnp
from jax import lax
from jax.experimental import pallas as pl
from jax.experimental.pallas import tpu as pltpu
from jax.experimental.pallas import tpu_sc as plsc

_NC = 2
_NS = 16
_NW = _NC * _NS
_CHUNK = 16
_NBUF = 4
_NCHUNK = 8


def _sc_copy(x_hbm, o_hbm, buf, in_sems, out_sems):
    wid = lax.axis_index("s") * _NC + lax.axis_index("c")
    base = wid * (_CHUNK * _NCHUNK)

    def in_copy(j, b):
        return pltpu.make_async_copy(
            x_hbm.at[pl.ds(base + j * _CHUNK, _CHUNK)], buf.at[b], in_sems.at[b]
        )

    def out_copy(j, b):
        return pltpu.make_async_copy(
            buf.at[b], o_hbm.at[pl.ds(base + j * _CHUNK, _CHUNK)], out_sems.at[b]
        )

    ins = [in_copy(j, j % _NBUF) for j in range(_NCHUNK)]
    outs = [out_copy(j, j % _NBUF) for j in range(_NCHUNK)]
    for j in range(_NBUF):
        ins[j].start()
    for j in range(_NCHUNK):
        ins[j].wait()
        outs[j].start()
        nxt = j + _NBUF
        if nxt < _NCHUNK:
            outs[j].wait()
            ins[nxt].start()
    for j in range(_NCHUNK - _NBUF, _NCHUNK):
        outs[j].wait()


def kernel(batch):
    B, F = batch.shape
    mesh = plsc.VectorSubcoreMesh(core_axis_name="c", subcore_axis_name="s")
    k = functools.partial(
        pl.kernel,
        mesh=mesh,
        out_type=jax.ShapeDtypeStruct((B, F), batch.dtype),
        scratch_types=[
            pltpu.VMEM((2, _CHUNK, F), batch.dtype),
            pltpu.SemaphoreType.DMA((2,)),
            pltpu.SemaphoreType.DMA((2,)),
        ],
    )(_sc_copy)
    return k(batch)


# speedup vs baseline: 2.8637x; 2.7678x over previous
import jax
import jax.numpy as jnp
from jax.experimental import pallas as pl
from jax.experimental.pallas import tpu as pltpu

_N = 8
_ROWS = 512


def _pipe_kernel(x_ref, o_ref, scratch, in_sems, out_sems):
    ins = [
        pltpu.make_async_copy(
            x_ref.at[pl.ds(i * _ROWS, _ROWS)], scratch.at[i], in_sems.at[i]
        )
        for i in range(_N)
    ]
    outs = [
        pltpu.make_async_copy(
            scratch.at[i], o_ref.at[pl.ds(i * _ROWS, _ROWS)], out_sems.at[i]
        )
        for i in range(_N)
    ]
    for c in ins:
        c.start()
    for i in range(_N):
        ins[i].wait()
        outs[i].start()
    for c in outs:
        c.wait()


def kernel(batch):
    B, F = batch.shape
    return pl.pallas_call(
        _pipe_kernel,
        in_specs=[pl.BlockSpec(memory_space=pl.ANY)],
        out_specs=pl.BlockSpec(memory_space=pl.ANY),
        out_shape=jax.ShapeDtypeStruct((B, F), batch.dtype),
        scratch_shapes=[
            pltpu.VMEM((_N, _ROWS, F), batch.dtype),
            pltpu.SemaphoreType.DMA((_N,)),
            pltpu.SemaphoreType.DMA((_N,)),
        ],
    )(batch)


# speedup vs baseline: 2.9562x; 1.0323x over previous
import jax
import jax.numpy as jnp
from jax.experimental import pallas as pl
from jax.experimental.pallas import tpu as pltpu


def _copy_kernel(x_ref, o_ref):
    o_ref[...] = x_ref[...]


def kernel(batch):
    B, F = batch.shape
    blk = 2048
    return pl.pallas_call(
        _copy_kernel,
        grid=(B // blk,),
        in_specs=[pl.BlockSpec((blk, F), lambda i: (i, 0))],
        out_specs=pl.BlockSpec((blk, F), lambda i: (i, 0)),
        out_shape=jax.ShapeDtypeStruct((B, F), batch.dtype),
        compiler_params=pltpu.CompilerParams(
            dimension_semantics=("arbitrary",),
        ),
    )(batch)
